# baseline (speedup 1.0000x reference)
import functools

import jax
import jax.numpy as jnp
from jax import lax
from jax.experimental import pallas as pl
from jax.experimental.pallas import tpu as pltpu
from jax.experimental.pallas import tpu_sc as plsc

NUM_EMB = 1000000
EMB_DIM = 64
SUB_DIM = 32
BASE_SEED = 42
BATCH = 4096
FIELDS = 26
N = BATCH * FIELDS
NW = 32
NPW = N // NW
CHUNK = 128
NCH = NPW // CHUNK
VPC = CHUNK // 16

_MAGIC = 1125899907
_MAGIC_HI = _MAGIC >> 16
_MAGIC_LO = _MAGIC & 0xFFFF


def _umod_1e6(h):
    al = h & jnp.uint32(0xFFFF)
    ah = h >> 16
    t = ah * jnp.uint32(_MAGIC_LO) + ((al * jnp.uint32(_MAGIC_LO)) >> 16)
    t2 = al * jnp.uint32(_MAGIC_HI) + (t & jnp.uint32(0xFFFF))
    hi = ah * jnp.uint32(_MAGIC_HI) + (t >> 16) + (t2 >> 16)
    q = hi >> 18
    return h - q * jnp.uint32(NUM_EMB)


def _hash16(v_u32, seed):
    h = v_u32 ^ jnp.uint32(seed)
    h = h * jnp.uint32(2654435761)
    h = h ^ (h >> 16)
    h = h * jnp.uint32(2246822519)
    h = h ^ (h >> 13)
    return plsc.bitcast(_umod_1e6(h), jnp.int32)


def _emb_body(ids_hbm, w0_hbm, w1_hbm, out_hbm,
              ids_v, idx0_v, idx1_v, buf0, buf1, sem0, sem1):
    wid = lax.axis_index("s") * 2 + lax.axis_index("c")
    base = wid * NPW

    pltpu.sync_copy(ids_hbm.at[pl.ds(base, NPW)], ids_v)

    def hash_chunk(c):
        for j in range(VPC):
            v = ids_v[pl.ds(c * CHUNK + j * 16, 16)]
            u = plsc.bitcast(v, jnp.uint32)
            idx0_v[c, pl.ds(j * 16, 16)] = _hash16(u, BASE_SEED)
            idx1_v[c, pl.ds(j * 16, 16)] = _hash16(u, BASE_SEED + 1)

    lax.fori_loop(0, NCH, lambda c, _: (hash_chunk(c), 0)[1], 0)

    def gather_chunk(c, _):
        cp0 = pltpu.async_copy(w0_hbm.at[idx0_v.at[c]], buf0, sem0)
        cp1 = pltpu.async_copy(w1_hbm.at[idx1_v.at[c]], buf1, sem1)
        cp0.wait()
        cp1.wait()
        row = base + c * CHUNK
        pltpu.sync_copy(buf0, out_hbm.at[pl.ds(row, CHUNK), pl.ds(0, SUB_DIM)])
        pltpu.sync_copy(buf1, out_hbm.at[pl.ds(row, CHUNK), pl.ds(SUB_DIM, SUB_DIM)])
        return 0

    lax.fori_loop(0, NCH, gather_chunk, 0)


_emb = functools.partial(
    pl.kernel,
    out_type=jax.ShapeDtypeStruct((N, EMB_DIM), jnp.float32),
    mesh=plsc.VectorSubcoreMesh(core_axis_name="c", subcore_axis_name="s"),
    compiler_params=pltpu.CompilerParams(use_tc_tiling_on_sc=False),
    scratch_types=[
        pltpu.VMEM((NPW,), jnp.int32),
        pltpu.VMEM((NCH, CHUNK), jnp.int32),
        pltpu.VMEM((NCH, CHUNK), jnp.int32),
        pltpu.VMEM((CHUNK, SUB_DIM), jnp.float32),
        pltpu.VMEM((CHUNK, SUB_DIM), jnp.float32),
        pltpu.SemaphoreType.DMA,
        pltpu.SemaphoreType.DMA,
    ],
)(_emb_body)


@jax.jit
def kernel(input_ids, W0, W1):
    flat = input_ids.reshape(-1)
    out = _emb(flat, W0, W1)
    return out.reshape(input_ids.shape + (EMB_DIM,))
